# speedup vs baseline: 3.4626x; 3.4626x over previous
import functools

import jax
import jax.numpy as jnp
from jax import lax
from jax.experimental import pallas as pl
from jax.experimental.pallas import tpu as pltpu
from jax.experimental.pallas import tpu_sc as plsc

D = 128
NC = 2
NS = 16
NW = NC * NS
CH = 128
NBUF = 4


@functools.lru_cache(maxsize=None)
def _build(N, V):
    NPW = N // NW
    G = NPW // CH
    mesh = plsc.VectorSubcoreMesh(core_axis_name="c", subcore_axis_name="s")

    @functools.partial(
        pl.kernel,
        out_type=jax.ShapeDtypeStruct((N, D), jnp.float32),
        mesh=mesh,
        scratch_types=[
            pltpu.VMEM((G, CH), jnp.int32),
            pltpu.VMEM((NBUF, CH, D), jnp.float32),
            pltpu.SemaphoreType.DMA,
            pltpu.SemaphoreType.DMA,
        ],
    )
    def emb_kernel(idx_hbm, emb_hbm, out_hbm, idx_v, bufs, gsem, wsem):
        wid = lax.axis_index("s") * NC + lax.axis_index("c")
        pltpu.sync_copy(idx_hbm.at[wid], idx_v)
        base = wid * NPW

        def group(o, carry):
            g0 = o * NBUF
            copies = []
            for b in range(NBUF):
                cp = pltpu.make_async_copy(
                    emb_hbm.at[idx_v.at[g0 + b]], bufs.at[b], gsem)
                cp.start()
                copies.append(cp)
            writes = []
            for b in range(NBUF):
                copies[b].wait()
                wr = pltpu.make_async_copy(
                    bufs.at[b],
                    out_hbm.at[pl.ds(base + (g0 + b) * CH, CH)],
                    wsem)
                wr.start()
                writes.append(wr)
            for wr in writes:
                wr.wait()
            return carry

        lax.fori_loop(0, G // NBUF, group, 0)

    return emb_kernel


def kernel(inputs, embedding):
    B, S = inputs.shape
    N = B * S
    V, d = embedding.shape
    idx = inputs.reshape(NW, (N // NW) // CH, CH).astype(jnp.int32)
    out = _build(N, V)(idx, embedding)
    return out.reshape(B, S, d)
